# Initial kernel scaffold; baseline (speedup 1.0000x reference)
import jax
import jax.numpy as jnp
from jax.experimental import pallas as pl


def kernel(features, edge_index, W1, b1, W2, b2, Wp, bp):
    raise NotImplementedError("write your pallas kernel here")


# speedup vs baseline: 3.2710x; 3.2710x over previous
import functools

import jax
import jax.numpy as jnp
from jax import lax
from jax.experimental import pallas as pl
from jax.experimental.pallas import tpu as pltpu
from jax.experimental.pallas import tpu_sc as plsc

_NC = 2
_NS = 16
_NW = _NC * _NS
_C = 80

_mesh = lambda: plsc.VectorSubcoreMesh(core_axis_name="c", subcore_axis_name="s")


def _tile_split(n_nodes):
    rt = -(-n_nodes // (8 * _NS)) * 8
    return rt, rt * _NS, n_nodes - rt * (_NS - 1)


def _sc_degrees(src, dst, n_nodes):
    e = src.shape[0]
    per_tile = e // _NW
    n_chunks = per_tile // _C
    rt, n_acc, tail = _tile_split(n_nodes)

    @functools.partial(
        pl.kernel,
        out_type=jax.ShapeDtypeStruct((_NC * n_nodes, 128), jnp.float32),
        mesh=_mesh(),
        scratch_types=[
            pltpu.VMEM((_C,), jnp.int32),
            pltpu.VMEM((_C,), jnp.int32),
            pltpu.VMEM((_C, 128), jnp.float32),
            pltpu.VMEM((_C, 128), jnp.float32),
            pltpu.VMEM_SHARED((n_acc, 128), jnp.float32),
        ],
    )
    def deg_kernel(src_h, dst_h, mark_s_h, mark_d_h, zeros_h, out_h,
                   sidx, didx, marks_v, markd_v, acc):
        c = lax.axis_index("c")
        s = lax.axis_index("s")
        wid = c * _NS + s
        pltpu.sync_copy(mark_s_h, marks_v)
        pltpu.sync_copy(mark_d_h, markd_v)
        pltpu.sync_copy(zeros_h, acc.at[pl.ds(s * rt, rt)])
        plsc.subcore_barrier()
        ebase = wid * per_tile

        def body(i, carry):
            b = ebase + i * _C
            pltpu.sync_copy(src_h.at[pl.ds(b, _C)], sidx)
            pltpu.sync_copy(dst_h.at[pl.ds(b, _C)], didx)
            pltpu.sync_copy(marks_v, acc.at[sidx], add=True)
            pltpu.sync_copy(markd_v, acc.at[didx], add=True)
            return carry

        lax.fori_loop(0, n_chunks, body, 0)
        plsc.subcore_barrier()

        @pl.when(s < _NS - 1)
        def _():
            oslc = pl.ds(c * n_nodes + s * rt, rt)
            pltpu.sync_copy(acc.at[pl.ds(s * rt, rt)], out_h.at[oslc])

        @pl.when(s == _NS - 1)
        def _():
            oslc = pl.ds(c * n_nodes + (_NS - 1) * rt, tail)
            pltpu.sync_copy(acc.at[pl.ds((_NS - 1) * rt, tail)], out_h.at[oslc])

    lane = lax.broadcasted_iota(jnp.int32, (_C, 128), 1)
    mark_s = jnp.where(lane < 64, 1.0, 0.0).astype(jnp.float32)
    mark_d = jnp.where(lane >= 64, 1.0, 0.0).astype(jnp.float32)
    zeros = jnp.zeros((rt, 128), jnp.float32)
    return deg_kernel(src, dst, mark_s, mark_d, zeros)


def _sc_segment_sum(table, src, dst):
    n_nodes, d = table.shape
    e = src.shape[0]
    per_tile = e // _NW
    n_chunks = per_tile // _C
    rt, n_acc, tail = _tile_split(n_nodes)

    @functools.partial(
        pl.kernel,
        out_type=jax.ShapeDtypeStruct((_NC * n_nodes, d), jnp.float32),
        mesh=_mesh(),
        scratch_types=[
            pltpu.VMEM((_C,), jnp.int32),
            pltpu.VMEM((_C,), jnp.int32),
            pltpu.VMEM((_C, d), jnp.float32),
            pltpu.VMEM_SHARED((n_acc, d), jnp.float32),
            pltpu.SemaphoreType.DMA,
        ],
    )
    def gs_kernel(table_h, src_h, dst_h, zeros_h, out_h,
                  sidx, didx, rows, acc, sem):
        c = lax.axis_index("c")
        s = lax.axis_index("s")
        wid = c * _NS + s
        pltpu.sync_copy(zeros_h, acc.at[pl.ds(s * rt, rt)])
        plsc.subcore_barrier()
        ebase = wid * per_tile

        def body(i, carry):
            b = ebase + i * _C
            pltpu.sync_copy(src_h.at[pl.ds(b, _C)], sidx)
            pltpu.sync_copy(dst_h.at[pl.ds(b, _C)], didx)
            pltpu.async_copy(table_h.at[sidx], rows, sem).wait()
            pltpu.sync_copy(rows, acc.at[didx], add=True)
            return carry

        lax.fori_loop(0, n_chunks, body, 0)
        plsc.subcore_barrier()

        @pl.when(s < _NS - 1)
        def _():
            oslc = pl.ds(c * n_nodes + s * rt, rt)
            pltpu.sync_copy(acc.at[pl.ds(s * rt, rt)], out_h.at[oslc])

        @pl.when(s == _NS - 1)
        def _():
            oslc = pl.ds(c * n_nodes + (_NS - 1) * rt, tail)
            pltpu.sync_copy(acc.at[pl.ds((_NS - 1) * rt, tail)], out_h.at[oslc])

    zeros = jnp.zeros((rt, d), jnp.float32)
    return gs_kernel(table, src, dst, zeros)


def _norms(d0, d1):
    deg = d0 + d1
    ns = lax.rsqrt(jnp.maximum(deg[:, 0:1], 1.0))
    nd = lax.rsqrt(jnp.maximum(deg[:, 64:65], 1.0))
    return ns, nd


def _tc_scale_src(features, deg, bn):
    n, d = features.shape
    nb = n // bn
    grid = (nb,)

    def body(f_ref, d0_ref, d1_ref, o_ref):
        ns, _ = _norms(d0_ref[...], d1_ref[...])
        o_ref[...] = f_ref[...] * ns

    return pl.pallas_call(
        body,
        grid=grid,
        in_specs=[
            pl.BlockSpec((bn, d), lambda i: (i, 0)),
            pl.BlockSpec((bn, 128), lambda i: (i, 0)),
            pl.BlockSpec((bn, 128), lambda i: (i + nb, 0)),
        ],
        out_specs=pl.BlockSpec((bn, d), lambda i: (i, 0)),
        out_shape=jax.ShapeDtypeStruct((n, d), jnp.float32),
    )(features, deg, deg)


def _tc_layer1(parts, deg, w1, b1, n, bn):
    d = parts.shape[1]
    hid = w1.shape[1]
    half = hid // 2
    nb = n // bn
    grid = (nb,)

    def body(p0_ref, p1_ref, d0_ref, d1_ref, w_ref, b_ref, oa_ref, ob_ref):
        ns, nd = _norms(d0_ref[...], d1_ref[...])
        agg = (p0_ref[...] + p1_ref[...]) * nd
        h = jnp.dot(agg, w_ref[...], preferred_element_type=jnp.float32)
        h = jnp.maximum(h + b_ref[...], 0.0)
        t = h * ns
        oa_ref[...] = t[:, :half]
        ob_ref[...] = t[:, half:]

    return pl.pallas_call(
        body,
        grid=grid,
        in_specs=[
            pl.BlockSpec((bn, d), lambda i: (i, 0)),
            pl.BlockSpec((bn, d), lambda i: (i + nb, 0)),
            pl.BlockSpec((bn, 128), lambda i: (i, 0)),
            pl.BlockSpec((bn, 128), lambda i: (i + nb, 0)),
            pl.BlockSpec((d, hid), lambda i: (0, 0)),
            pl.BlockSpec((1, hid), lambda i: (0, 0)),
        ],
        out_specs=[
            pl.BlockSpec((bn, half), lambda i: (i, 0)),
            pl.BlockSpec((bn, half), lambda i: (i, 0)),
        ],
        out_shape=[jax.ShapeDtypeStruct((n, half), jnp.float32),
                   jax.ShapeDtypeStruct((n, half), jnp.float32)],
    )(parts, parts, deg, deg, w1, b1)


def _tc_layer2_head(parts_a, parts_b, deg, w2, b2, wp, bp, n, bn):
    half = parts_a.shape[1]
    hid = w2.shape[0]
    n_out = wp.shape[1]
    nb = n // bn
    grid = (nb,)

    def body(pa0_ref, pa1_ref, pb0_ref, pb1_ref, d0_ref, d1_ref,
             w_ref, b_ref, wp_ref, bp_ref, o_ref, acc_ref):
        i = pl.program_id(0)
        _, nd = _norms(d0_ref[...], d1_ref[...])
        agg = jnp.concatenate(
            [pa0_ref[...] + pa1_ref[...], pb0_ref[...] + pb1_ref[...]],
            axis=1) * nd
        h = jnp.dot(agg, w_ref[...], preferred_element_type=jnp.float32)
        h = jnp.maximum(h + b_ref[...], 0.0)
        part = jnp.sum(h, axis=0, keepdims=True)

        @pl.when(i == 0)
        def _():
            acc_ref[...] = part

        @pl.when(i > 0)
        def _():
            acc_ref[...] = acc_ref[...] + part

        @pl.when(i == nb - 1)
        def _():
            hg = acc_ref[...] * (1.0 / n)
            o_ref[...] = jnp.dot(hg, wp_ref[...],
                                 preferred_element_type=jnp.float32) + bp_ref[...]

    return pl.pallas_call(
        body,
        grid=grid,
        in_specs=[
            pl.BlockSpec((bn, half), lambda i: (i, 0)),
            pl.BlockSpec((bn, half), lambda i: (i + nb, 0)),
            pl.BlockSpec((bn, half), lambda i: (i, 0)),
            pl.BlockSpec((bn, half), lambda i: (i + nb, 0)),
            pl.BlockSpec((bn, 128), lambda i: (i, 0)),
            pl.BlockSpec((bn, 128), lambda i: (i + nb, 0)),
            pl.BlockSpec((hid, hid), lambda i: (0, 0)),
            pl.BlockSpec((1, hid), lambda i: (0, 0)),
            pl.BlockSpec((hid, n_out), lambda i: (0, 0)),
            pl.BlockSpec((1, n_out), lambda i: (0, 0)),
        ],
        out_specs=pl.BlockSpec((1, n_out), lambda i: (0, 0)),
        out_shape=jax.ShapeDtypeStruct((1, n_out), jnp.float32),
        scratch_shapes=[pltpu.VMEM((1, hid), jnp.float32)],
    )(parts_a, parts_a, parts_b, parts_b, deg, deg, w2, b2, wp, bp)


def kernel(features, edge_index, W1, b1, W2, b2, Wp, bp):
    n, d = features.shape
    src = edge_index[0]
    dst = edge_index[1]
    bn = 1000

    deg = _sc_degrees(src, dst, n)
    table1 = _tc_scale_src(features, deg, bn)
    parts1 = _sc_segment_sum(table1, src, dst)
    t2a, t2b = _tc_layer1(parts1, deg,
                          W1, b1.reshape(1, -1), n, bn)
    parts2a = _sc_segment_sum(t2a, src, dst)
    parts2b = _sc_segment_sum(t2b, src, dst)
    return _tc_layer2_head(parts2a, parts2b, deg,
                           W2, b2.reshape(1, -1),
                           Wp, bp.reshape(1, -1), n, bn)


# speedup vs baseline: 4.6948x; 1.4353x over previous
import functools

import jax
import jax.numpy as jnp
from jax import lax
from jax.experimental import pallas as pl
from jax.experimental.pallas import tpu as pltpu
from jax.experimental.pallas import tpu_sc as plsc

_NC = 2
_NS = 16
_NW = _NC * _NS
_C = 80

_mesh = lambda: plsc.VectorSubcoreMesh(core_axis_name="c", subcore_axis_name="s")


def _tile_split(n_nodes):
    rt = -(-n_nodes // (8 * _NS)) * 8
    return rt, rt * _NS, n_nodes - rt * (_NS - 1)


def _sc_degrees(src, dst, n_nodes):
    e = src.shape[0]
    per_tile = e // _NW
    cc = 40
    n_chunks = per_tile // cc
    n_pairs = n_chunks // 2
    assert n_chunks % 2 == 0
    rt, n_acc, tail = _tile_split(n_nodes)

    @functools.partial(
        pl.kernel,
        out_type=jax.ShapeDtypeStruct((_NC * n_nodes, 128), jnp.float32),
        mesh=_mesh(),
        scratch_types=[
            [pltpu.VMEM((cc,), jnp.int32) for _ in range(2)],
            [pltpu.VMEM((cc,), jnp.int32) for _ in range(2)],
            pltpu.VMEM((cc, 128), jnp.float32),
            pltpu.VMEM((cc, 128), jnp.float32),
            pltpu.VMEM_SHARED((n_acc, 128), jnp.float32),
            [pltpu.SemaphoreType.DMA for _ in range(2)],
        ],
    )
    def deg_kernel(src_h, dst_h, mark_s_h, mark_d_h, zeros_h, out_h,
                   sidx, didx, marks_v, markd_v, acc, isem):
        c = lax.axis_index("c")
        s = lax.axis_index("s")
        wid = c * _NS + s
        ebase = wid * per_tile
        pltpu.sync_copy(mark_s_h, marks_v)
        pltpu.sync_copy(mark_d_h, markd_v)
        pltpu.sync_copy(zeros_h, acc.at[pl.ds(s * rt, rt)])
        plsc.subcore_barrier()

        def fire_idx(i, b):
            pltpu.async_copy(src_h.at[pl.ds(ebase + i * cc, cc)],
                             sidx[b], isem[b])
            pltpu.async_copy(dst_h.at[pl.ds(ebase + i * cc, cc)],
                             didx[b], isem[b])

        def wait_idx(b):
            pltpu.make_async_copy(src_h.at[pl.ds(0, cc)], sidx[b],
                                  isem[b]).wait()
            pltpu.make_async_copy(src_h.at[pl.ds(0, cc)], didx[b],
                                  isem[b]).wait()

        for b in range(2):
            fire_idx(b, b)

        def body(j, carry):
            for b in range(2):
                wait_idx(b)
                pltpu.sync_copy(marks_v, acc.at[sidx[b]], add=True)
                pltpu.sync_copy(markd_v, acc.at[didx[b]], add=True)
                fire_idx(2 * (j + 1) + b, b)
            return carry

        lax.fori_loop(0, n_pairs - 1, body, 0)
        for b in range(2):
            wait_idx(b)
            pltpu.sync_copy(marks_v, acc.at[sidx[b]], add=True)
            pltpu.sync_copy(markd_v, acc.at[didx[b]], add=True)
        plsc.subcore_barrier()

        @pl.when(s < _NS - 1)
        def _():
            oslc = pl.ds(c * n_nodes + s * rt, rt)
            pltpu.sync_copy(acc.at[pl.ds(s * rt, rt)], out_h.at[oslc])

        @pl.when(s == _NS - 1)
        def _():
            oslc = pl.ds(c * n_nodes + (_NS - 1) * rt, tail)
            pltpu.sync_copy(acc.at[pl.ds((_NS - 1) * rt, tail)], out_h.at[oslc])

    lane = lax.broadcasted_iota(jnp.int32, (cc, 128), 1)
    mark_s = jnp.where(lane < 64, 1.0, 0.0).astype(jnp.float32)
    mark_d = jnp.where(lane >= 64, 1.0, 0.0).astype(jnp.float32)
    zeros = jnp.zeros((rt, 128), jnp.float32)
    return deg_kernel(src, dst, mark_s, mark_d, zeros)


def _sc_segment_sum(table, src, dst):
    n_nodes, d = table.shape
    e = src.shape[0]
    per_tile = e // _NW
    cc = 40
    n_chunks = per_tile // cc
    n_pairs = n_chunks // 2
    assert n_chunks % 2 == 0
    rt, n_acc, tail = _tile_split(n_nodes)

    @functools.partial(
        pl.kernel,
        out_type=jax.ShapeDtypeStruct((_NC * n_nodes, d), jnp.float32),
        mesh=_mesh(),
        scratch_types=[
            [pltpu.VMEM((cc,), jnp.int32) for _ in range(2)],
            [pltpu.VMEM((cc,), jnp.int32) for _ in range(2)],
            [pltpu.VMEM((cc, d), jnp.float32) for _ in range(2)],
            pltpu.VMEM_SHARED((n_acc, d), jnp.float32),
            [pltpu.SemaphoreType.DMA for _ in range(2)],
            [pltpu.SemaphoreType.DMA for _ in range(2)],
        ],
    )
    def gs_kernel(table_h, src_h, dst_h, zeros_h, out_h,
                  sidx, didx, rows, acc, isem, gsem):
        c = lax.axis_index("c")
        s = lax.axis_index("s")
        wid = c * _NS + s
        ebase = wid * per_tile
        pltpu.sync_copy(zeros_h, acc.at[pl.ds(s * rt, rt)])
        plsc.subcore_barrier()

        def fire_idx(i, b):
            pltpu.async_copy(src_h.at[pl.ds(ebase + i * cc, cc)],
                             sidx[b], isem[b])
            pltpu.async_copy(dst_h.at[pl.ds(ebase + i * cc, cc)],
                             didx[b], isem[b])

        def wait_idx(b):
            pltpu.make_async_copy(src_h.at[pl.ds(0, cc)], sidx[b],
                                  isem[b]).wait()
            pltpu.make_async_copy(src_h.at[pl.ds(0, cc)], didx[b],
                                  isem[b]).wait()

        def wait_gather(b):
            pltpu.make_async_copy(zeros_h.at[pl.ds(0, cc)], rows[b],
                                  gsem[b]).wait()

        for b in range(2):
            fire_idx(b, b)

        def body(j, carry):
            for b in range(2):
                wait_idx(b)
                pltpu.async_copy(table_h.at[sidx[b]], rows[b], gsem[b])
            for b in range(2):
                wait_gather(b)
                pltpu.sync_copy(rows[b], acc.at[didx[b]], add=True)
                fire_idx(2 * (j + 1) + b, b)
            return carry

        lax.fori_loop(0, n_pairs - 1, body, 0)
        for b in range(2):
            wait_idx(b)
            pltpu.async_copy(table_h.at[sidx[b]], rows[b], gsem[b])
        for b in range(2):
            wait_gather(b)
            pltpu.sync_copy(rows[b], acc.at[didx[b]], add=True)
        plsc.subcore_barrier()

        @pl.when(s < _NS - 1)
        def _():
            oslc = pl.ds(c * n_nodes + s * rt, rt)
            pltpu.sync_copy(acc.at[pl.ds(s * rt, rt)], out_h.at[oslc])

        @pl.when(s == _NS - 1)
        def _():
            oslc = pl.ds(c * n_nodes + (_NS - 1) * rt, tail)
            pltpu.sync_copy(acc.at[pl.ds((_NS - 1) * rt, tail)], out_h.at[oslc])

    zeros = jnp.zeros((rt, d), jnp.float32)
    return gs_kernel(table, src, dst, zeros)


def _norms(d0, d1):
    deg = d0 + d1
    ns = lax.rsqrt(jnp.maximum(deg[:, 0:1], 1.0))
    nd = lax.rsqrt(jnp.maximum(deg[:, 64:65], 1.0))
    return ns, nd


def _tc_scale_src(features, deg, bn):
    n, d = features.shape
    nb = n // bn
    grid = (nb,)

    def body(f_ref, d0_ref, d1_ref, o_ref):
        ns, _ = _norms(d0_ref[...], d1_ref[...])
        o_ref[...] = f_ref[...] * ns

    return pl.pallas_call(
        body,
        grid=grid,
        in_specs=[
            pl.BlockSpec((bn, d), lambda i: (i, 0)),
            pl.BlockSpec((bn, 128), lambda i: (i, 0)),
            pl.BlockSpec((bn, 128), lambda i: (i + nb, 0)),
        ],
        out_specs=pl.BlockSpec((bn, d), lambda i: (i, 0)),
        out_shape=jax.ShapeDtypeStruct((n, d), jnp.float32),
    )(features, deg, deg)


def _tc_layer1(parts, deg, w1, b1, n, bn):
    d = parts.shape[1]
    hid = w1.shape[1]
    half = hid // 2
    nb = n // bn
    grid = (nb,)

    def body(p0_ref, p1_ref, d0_ref, d1_ref, w_ref, b_ref, oa_ref, ob_ref):
        ns, nd = _norms(d0_ref[...], d1_ref[...])
        agg = (p0_ref[...] + p1_ref[...]) * nd
        h = jnp.dot(agg, w_ref[...], preferred_element_type=jnp.float32)
        h = jnp.maximum(h + b_ref[...], 0.0)
        t = h * ns
        oa_ref[...] = t[:, :half]
        ob_ref[...] = t[:, half:]

    return pl.pallas_call(
        body,
        grid=grid,
        in_specs=[
            pl.BlockSpec((bn, d), lambda i: (i, 0)),
            pl.BlockSpec((bn, d), lambda i: (i + nb, 0)),
            pl.BlockSpec((bn, 128), lambda i: (i, 0)),
            pl.BlockSpec((bn, 128), lambda i: (i + nb, 0)),
            pl.BlockSpec((d, hid), lambda i: (0, 0)),
            pl.BlockSpec((1, hid), lambda i: (0, 0)),
        ],
        out_specs=[
            pl.BlockSpec((bn, half), lambda i: (i, 0)),
            pl.BlockSpec((bn, half), lambda i: (i, 0)),
        ],
        out_shape=[jax.ShapeDtypeStruct((n, half), jnp.float32),
                   jax.ShapeDtypeStruct((n, half), jnp.float32)],
    )(parts, parts, deg, deg, w1, b1)


def _tc_layer2_head(parts_a, parts_b, deg, w2, b2, wp, bp, n, bn):
    half = parts_a.shape[1]
    hid = w2.shape[0]
    n_out = wp.shape[1]
    nb = n // bn
    grid = (nb,)

    def body(pa0_ref, pa1_ref, pb0_ref, pb1_ref, d0_ref, d1_ref,
             w_ref, b_ref, wp_ref, bp_ref, o_ref, acc_ref):
        i = pl.program_id(0)
        _, nd = _norms(d0_ref[...], d1_ref[...])
        agg = jnp.concatenate(
            [pa0_ref[...] + pa1_ref[...], pb0_ref[...] + pb1_ref[...]],
            axis=1) * nd
        h = jnp.dot(agg, w_ref[...], preferred_element_type=jnp.float32)
        h = jnp.maximum(h + b_ref[...], 0.0)
        part = jnp.sum(h, axis=0, keepdims=True)

        @pl.when(i == 0)
        def _():
            acc_ref[...] = part

        @pl.when(i > 0)
        def _():
            acc_ref[...] = acc_ref[...] + part

        @pl.when(i == nb - 1)
        def _():
            hg = acc_ref[...] * (1.0 / n)
            o_ref[...] = jnp.dot(hg, wp_ref[...],
                                 preferred_element_type=jnp.float32) + bp_ref[...]

    return pl.pallas_call(
        body,
        grid=grid,
        in_specs=[
            pl.BlockSpec((bn, half), lambda i: (i, 0)),
            pl.BlockSpec((bn, half), lambda i: (i + nb, 0)),
            pl.BlockSpec((bn, half), lambda i: (i, 0)),
            pl.BlockSpec((bn, half), lambda i: (i + nb, 0)),
            pl.BlockSpec((bn, 128), lambda i: (i, 0)),
            pl.BlockSpec((bn, 128), lambda i: (i + nb, 0)),
            pl.BlockSpec((hid, hid), lambda i: (0, 0)),
            pl.BlockSpec((1, hid), lambda i: (0, 0)),
            pl.BlockSpec((hid, n_out), lambda i: (0, 0)),
            pl.BlockSpec((1, n_out), lambda i: (0, 0)),
        ],
        out_specs=pl.BlockSpec((1, n_out), lambda i: (0, 0)),
        out_shape=jax.ShapeDtypeStruct((1, n_out), jnp.float32),
        scratch_shapes=[pltpu.VMEM((1, hid), jnp.float32)],
    )(parts_a, parts_a, parts_b, parts_b, deg, deg, w2, b2, wp, bp)


def kernel(features, edge_index, W1, b1, W2, b2, Wp, bp):
    n, d = features.shape
    src = edge_index[0]
    dst = edge_index[1]
    bn = 1000

    deg = _sc_degrees(src, dst, n)
    table1 = _tc_scale_src(features, deg, bn)
    parts1 = _sc_segment_sum(table1, src, dst)
    t2a, t2b = _tc_layer1(parts1, deg,
                          W1, b1.reshape(1, -1), n, bn)
    parts2a = _sc_segment_sum(t2a, src, dst)
    parts2b = _sc_segment_sum(t2b, src, dst)
    return _tc_layer2_head(parts2a, parts2b, deg,
                           W2, b2.reshape(1, -1),
                           Wp, bp.reshape(1, -1), n, bn)
